# Initial kernel scaffold; baseline (speedup 1.0000x reference)
import jax
import jax.numpy as jnp
from jax.experimental import pallas as pl


def kernel(x, positional_embedding, token_embedding):
    raise NotImplementedError("write your pallas kernel here")


# speedup vs baseline: 1.1267x; 1.1267x over previous
import functools

import jax
import jax.numpy as jnp
from jax import lax
from jax.experimental import pallas as pl
from jax.experimental.pallas import tpu as pltpu
from jax.experimental.pallas import tpu_sc as plsc

BATCH = 4
SEQ = 2048
WIDTH = 512
NUM_CORES = 2
NUM_SUBCORES = 16
NUM_WORKERS = NUM_CORES * NUM_SUBCORES
S_PER_W = SEQ // NUM_WORKERS
LANES = 16
CHUNKS = WIDTH // LANES


def _sc_kernel(x_hbm, pos_hbm, table_hbm, out_hbm,
               idx_v, pos_v, rows0, rows1, gsem):
    wid = lax.axis_index("s") * NUM_CORES + lax.axis_index("c")
    s_base = wid * S_PER_W

    for b in range(BATCH):
        pltpu.sync_copy(x_hbm.at[pl.ds(b * SEQ + s_base, S_PER_W)],
                        idx_v.at[b])

    rows = (rows0, rows1)
    copies = [pltpu.async_copy(table_hbm.at[idx_v.at[0]], rows[0], gsem)]

    pltpu.sync_copy(pos_hbm.at[pl.ds(s_base, S_PER_W)], pos_v)

    for b in range(BATCH):
        buf = rows[b % 2]
        copies[b].wait()
        if b + 1 < BATCH:
            copies.append(
                pltpu.async_copy(table_hbm.at[idx_v.at[b + 1]],
                                 rows[(b + 1) % 2], gsem))

        def add_row(i, _, buf=buf):
            for j in range(CHUNKS):
                sl = pl.ds(j * LANES, LANES)
                buf[i, sl] = buf[i, sl] + pos_v[i, sl]
            return _

        lax.fori_loop(0, S_PER_W, add_row, None)
        pltpu.sync_copy(buf, out_hbm.at[pl.ds(b * SEQ + s_base, S_PER_W)])


@jax.jit
def _gpt_pre_encode(xf, positional_embedding, token_embedding):
    mesh = plsc.VectorSubcoreMesh(core_axis_name="c", subcore_axis_name="s",
                                  num_cores=NUM_CORES,
                                  num_subcores=NUM_SUBCORES)
    run = pl.kernel(
        _sc_kernel,
        out_type=jax.ShapeDtypeStruct((BATCH * SEQ, WIDTH), jnp.float32),
        mesh=mesh,
        scratch_types=[
            pltpu.VMEM((BATCH, S_PER_W), jnp.int32),
            pltpu.VMEM((S_PER_W, WIDTH), jnp.float32),
            pltpu.VMEM((S_PER_W, WIDTH), jnp.float32),
            pltpu.VMEM((S_PER_W, WIDTH), jnp.float32),
            pltpu.SemaphoreType.DMA,
        ],
    )
    return run(xf, positional_embedding, token_embedding)


def kernel(x, positional_embedding, token_embedding):
    xf = x.reshape(BATCH * SEQ).astype(jnp.int32)
    out = _gpt_pre_encode(xf, positional_embedding, token_embedding)
    return out.reshape(BATCH, SEQ, WIDTH)
